# baseline (speedup 1.0000x reference)
import jax
import jax.numpy as jnp
from jax import lax
from jax.experimental import pallas as pl
from jax.experimental.pallas import tpu as pltpu
from jax.experimental.pallas import tpu_sc as plsc

N = 10000
E = 160000
M = 2 * E
NC, NS, LANES = 2, 16, 16
NW = NC * NS
NPAD = 10240
NPB = NPAD // NW
EPT = 10240
MPAD = NW * EPT
CHUNK = 2048
NCHUNK = EPT // CHUNK
STEPS = CHUNK // LANES
D = 128
PLANE = NPAD * D + NW * LANES
NB = 8
GRID = NPAD // NB

ANGLE_MIN = 100.0
SCALE = 0.001
RAD2DEG = 57.29577951308232
_ACOS_C = (
    1.4142131394e+00, 1.1789074826e-01, 2.5914469662e-02, 1.1320778884e-02,
    -6.6692835552e-03, 1.4279976500e-02, -9.2624280065e-03, 3.1091512747e-03,
)
CLIP_LO = -0.3035

_MESH = dict(core_axis_name="c", subcore_axis_name="s")


def _wid():
    return lax.axis_index("s") * NC + lax.axis_index("c")


def _zero_i32(ref, n):
    def zb(i, _):
        ref[pl.ds(i * LANES, LANES)] = jnp.zeros((LANES,), jnp.int32)
        return 0
    lax.fori_loop(0, n // LANES, zb, 0)


def _sc_hist_body(srcs, dsts, ets, hist_out, hv, sv, dv, ev, sem):
    w = _wid()
    _zero_i32(hv, NPAD)

    def chunk(ci, _):
        base = w * EPT + ci * CHUNK
        pltpu.async_copy(srcs.at[pl.ds(base, CHUNK)], sv, sem).wait()
        pltpu.async_copy(dsts.at[pl.ds(base, CHUNK)], dv, sem).wait()
        pltpu.async_copy(ets.at[pl.ds(base, CHUNK)], ev, sem).wait()

        def step(i, _):
            s = sv[pl.ds(i * LANES, LANES)]
            d = dv[pl.ds(i * LANES, LANES)]
            et = ev[pl.ds(i * LANES, LANES)]
            valid = (et != 0) & (s != d)
            r, last = plsc.scan_count(s, mask=valid)
            basec = plsc.load_gather(hv, [s])
            plsc.store_scatter(hv, [s], basec + r, mask=valid & last)
            return 0

        lax.fori_loop(0, STEPS, step, 0)
        return 0

    lax.fori_loop(0, NCHUNK, chunk, 0)
    pltpu.sync_copy(hv, hist_out.at[pl.ds(w * NPAD, NPAD)])


def _sc_prefix_body(hist_in, tb_out, cnt_out, hv, tv, av, sem):
    w = _wid()
    descs = [
        pltpu.async_copy(
            hist_in.at[pl.ds(t * NPAD + w * NPB, NPB)], hv.at[pl.ds(t * NPB, NPB)], sem
        )
        for t in range(NW)
    ]
    for de in descs:
        de.wait()
    _zero_i32(av, NPB)
    for t in range(NW):
        def stepj(j, _, t=t):
            a = av[pl.ds(j * LANES, LANES)]
            h = hv[pl.ds(t * NPB + j * LANES, LANES)]
            tv[pl.ds(t * NPB + j * LANES, LANES)] = a
            av[pl.ds(j * LANES, LANES)] = a + h
            return 0
        lax.fori_loop(0, NPB // LANES, stepj, 0)
    descs = [
        pltpu.async_copy(
            tv.at[pl.ds(t * NPB, NPB)], tb_out.at[pl.ds(t * NPAD + w * NPB, NPB)], sem
        )
        for t in range(NW)
    ]
    for de in descs:
        de.wait()
    pltpu.sync_copy(av, cnt_out.at[pl.ds(w * NPB, NPB)])


def _sc_scatter_body(srcs, dsts, ets, xt, tb_in, p0, p1, p2, p3,
                     x0v, x1v, x2v, tbv, cv, sv, dv, ev,
                     six, s0, s1, s2, s3, sem, semo):
    w = _wid()
    pltpu.async_copy(xt.at[pl.ds(0, NPAD)], x0v, sem).wait()
    pltpu.async_copy(xt.at[pl.ds(NPAD, NPAD)], x1v, sem).wait()
    pltpu.async_copy(xt.at[pl.ds(2 * NPAD, NPAD)], x2v, sem).wait()
    pltpu.async_copy(tb_in.at[pl.ds(w * NPAD, NPAD)], tbv, sem).wait()
    _zero_i32(cv, NPAD)
    lane = lax.iota(jnp.int32, LANES)
    trash0 = NPAD * D + w * LANES

    def chunk(ci, _):
        base = w * EPT + ci * CHUNK
        pltpu.async_copy(srcs.at[pl.ds(base, CHUNK)], sv, sem).wait()
        pltpu.async_copy(dsts.at[pl.ds(base, CHUNK)], dv, sem).wait()
        pltpu.async_copy(ets.at[pl.ds(base, CHUNK)], ev, sem).wait()

        def grp(g, _):
            for k in range(STEPS // LANES):
                i = g * (STEPS // LANES) + k
                s = sv[pl.ds(i * LANES, LANES)]
                d = dv[pl.ds(i * LANES, LANES)]
                et = ev[pl.ds(i * LANES, LANES)]
                valid = (et != 0) & (s != d)
                r, last = plsc.scan_count(s, mask=valid)
                cg = plsc.load_gather(cv, [s])
                plsc.store_scatter(cv, [s], cg + r, mask=valid & last)
                tbg = plsc.load_gather(tbv, [s])
                slot = tbg + cg + (r - 1)
                flat = s * D + slot
                flat = jnp.where(valid, flat, trash0 + lane)
                six[g, pl.ds(k * LANES, LANES)] = flat
                xs = plsc.load_gather(x0v, [s])
                xd = plsc.load_gather(x0v, [d])
                s0[pl.ds(i * LANES, LANES)] = xd - xs
                xs = plsc.load_gather(x1v, [s])
                xd = plsc.load_gather(x1v, [d])
                s1[pl.ds(i * LANES, LANES)] = xd - xs
                xs = plsc.load_gather(x2v, [s])
                xd = plsc.load_gather(x2v, [d])
                s2[pl.ds(i * LANES, LANES)] = xd - xs
                s3[pl.ds(i * LANES, LANES)] = d.astype(jnp.float32)
            return 0

        lax.fori_loop(0, LANES, grp, 0)
        descs = []
        for j in range(LANES):
            sl = pl.ds(j * 128, 128)
            ix = six.at[j]
            descs.append(pltpu.async_copy(s0.at[sl], p0.at[ix], semo))
            descs.append(pltpu.async_copy(s1.at[sl], p1.at[ix], semo))
            descs.append(pltpu.async_copy(s2.at[sl], p2.at[ix], semo))
            descs.append(pltpu.async_copy(s3.at[sl], p3.at[ix], semo))
        for de in descs:
            de.wait()
        return 0

    lax.fori_loop(0, NCHUNK, chunk, 0)


def _make_sc_kernels():
    mesh = plsc.VectorSubcoreMesh(**_MESH)
    i32 = jnp.int32
    params = pltpu.CompilerParams(needs_layout_passes=False)
    hist = pl.kernel(
        _sc_hist_body,
        out_type=jax.ShapeDtypeStruct((NW * NPAD,), i32),
        mesh=mesh,
        compiler_params=params,
        scratch_types=[
            pltpu.VMEM((NPAD,), i32), pltpu.VMEM((CHUNK,), i32),
            pltpu.VMEM((CHUNK,), i32), pltpu.VMEM((CHUNK,), i32),
            pltpu.SemaphoreType.DMA,
        ],
    )
    prefix = pl.kernel(
        _sc_prefix_body,
        out_type=(
            jax.ShapeDtypeStruct((NW * NPAD,), i32),
            jax.ShapeDtypeStruct((NPAD,), i32),
        ),
        mesh=mesh,
        scratch_types=[
            pltpu.VMEM((NPAD,), i32), pltpu.VMEM((NPAD,), i32),
            pltpu.VMEM((NPB,), i32), pltpu.SemaphoreType.DMA,
        ],
    )
    f32 = jnp.float32
    scatter = pl.kernel(
        _sc_scatter_body,
        out_type=tuple(jax.ShapeDtypeStruct((PLANE,), f32) for _ in range(4)),
        mesh=mesh,
        compiler_params=params,
        scratch_types=[
            pltpu.VMEM((NPAD,), f32), pltpu.VMEM((NPAD,), f32),
            pltpu.VMEM((NPAD,), f32), pltpu.VMEM((NPAD,), i32),
            pltpu.VMEM((NPAD,), i32), pltpu.VMEM((CHUNK,), i32),
            pltpu.VMEM((CHUNK,), i32), pltpu.VMEM((CHUNK,), i32),
            pltpu.VMEM((LANES, 128), i32),
            pltpu.VMEM((CHUNK,), f32), pltpu.VMEM((CHUNK,), f32),
            pltpu.VMEM((CHUNK,), f32), pltpu.VMEM((CHUNK,), f32),
            pltpu.SemaphoreType.DMA, pltpu.SemaphoreType.DMA,
        ],
    )
    return hist, prefix, scatter


def _tc_pair_body(p0, p1, p2, p3, cref, acc_ref, out_ref):
    pid = pl.program_id(0)

    @pl.when(pid == 0)
    def _():
        acc_ref[...] = jnp.zeros_like(acc_ref)

    c = cref[...]
    v0 = p0[...]
    v1 = p1[...]
    v2 = p2[...]
    dn = p3[...]
    jl = lax.broadcasted_iota(jnp.int32, (NB, D), 1)
    validj = jl < c
    n2 = v0 * v0 + v1 * v1 + v2 * v2
    inv = lax.rsqrt(jnp.maximum(n2, 1e-30))
    u0 = jnp.where(validj, v0 * inv, 0.0)
    u1 = jnp.where(validj, v1 * inv, 0.0)
    u2 = jnp.where(validj, v2 * inv, 0.0)
    cmax = jnp.max(c)

    def off(o, acc):
        sh = D - o
        r0 = pltpu.roll(u0, sh, 1)
        r1 = pltpu.roll(u1, sh, 1)
        r2 = pltpu.roll(u2, sh, 1)
        rd = pltpu.roll(dn, sh, 1)
        cos = u0 * r0 + u1 * r1 + u2 * r2
        pairm = ((jl + o) < c) & (dn != rd)
        s = 1.0 - jnp.clip(cos, CLIP_LO, 1.0)
        q = jnp.float32(_ACOS_C[7])
        for coef in _ACOS_C[6::-1]:
            q = q * s + jnp.float32(coef)
        ang = RAD2DEG * jnp.sqrt(s) * q
        drift = jnp.maximum(ANGLE_MIN - ang, 0.0)
        return acc + jnp.where(pairm, drift, 0.0)

    acc = lax.fori_loop(1, cmax, off, jnp.zeros((NB, D), jnp.float32))
    acc_ref[...] += acc

    @pl.when(pid == GRID - 1)
    def _():
        out_ref[...] = (jnp.sum(acc_ref[...]) * SCALE).reshape(1, 1)


def _tc_pairs(q0, q1, q2, q3, cnt2d):
    spec = pl.BlockSpec((NB, D), lambda g: (g, 0))
    cspec = pl.BlockSpec((NB, 1), lambda g: (g, 0))
    accspec = pl.BlockSpec((NB, D), lambda g: (0, 0))
    outspec = pl.BlockSpec((1, 1), lambda g: (0, 0))
    acc, out = pl.pallas_call(
        _tc_pair_body,
        grid=(GRID,),
        in_specs=[spec, spec, spec, spec, cspec],
        out_specs=[accspec, outspec],
        out_shape=[
            jax.ShapeDtypeStruct((NB, D), jnp.float32),
            jax.ShapeDtypeStruct((1, 1), jnp.float32),
        ],
    )(q0, q1, q2, q3, cnt2d)
    return out


def kernel(x, e_type, e_index):
    i32 = jnp.int32
    e0 = e_index[0].astype(i32)
    e1 = e_index[1].astype(i32)
    et = e_type.astype(i32)
    padm = jnp.zeros((MPAD - M,), i32)
    srcs = jnp.concatenate([e0, e1, padm])
    dsts = jnp.concatenate([e1, e0, padm])
    ets = jnp.concatenate([et, et, padm])
    xt = jnp.pad(x.astype(jnp.float32).T, ((0, 0), (0, NPAD - N))).reshape(-1)

    hist_k, prefix_k, scatter_k = _make_sc_kernels()
    hist = hist_k(srcs, dsts, ets)
    tb, cnt = prefix_k(hist)
    p0, p1, p2, p3 = scatter_k(srcs, dsts, ets, xt, tb)

    q = [jnp.reshape(p, (PLANE // D, D)) for p in (p0, p1, p2, p3)]
    cnt2d = jnp.reshape(cnt, (NPAD, 1))
    out = _tc_pairs(*q, cnt2d)
    return out[0, 0]
